# baseline (device time: 18341 ns/iter reference)
import jax
import jax.numpy as jnp
from jax import lax
from jax.experimental import pallas as pl
from jax.experimental.pallas import tpu as pltpu

N_DEV = 4


def kernel(x):
    m, n = x.shape
    q = m // 4
    h = m // 2
    o = q // 2
    cw = n // 2

    CA = pl.ds(0, cw)
    CB = pl.ds(cw, cw)

    def body(x_ref, out_ref, r1A, r1B, r2A, r2B,
             send_sems, recv_sems):
        my = lax.axis_index("i")
        g = my // 2
        b = lax.rem(lax.rem(my, 2) + g, 2)
        p_b = jnp.bitwise_xor(my, 1)
        p_g = jnp.bitwise_xor(my, 3)

        k_own = 2 * g + b
        k_pb = 2 * g + 1 - b
        k_pg = 2 * (1 - g) + b
        k_dg = 2 * (1 - g) + 1 - b
        hk = pl.ds(g * h, h)

        def oct_(k, j):
            return pl.ds(k * q + j * o, o)

        def bf(row_slice, col_slice):
            return x_ref[row_slice, col_slice]

        barrier_sem = pltpu.get_barrier_semaphore()
        for nbr in (p_b, p_g):
            pl.semaphore_signal(
                barrier_sem, inc=1,
                device_id=(nbr,), device_id_type=pl.DeviceIdType.MESH,
            )
        pl.semaphore_wait(barrier_sem, 2)

        def rdma(src, dst, ch, dev):
            return pltpu.make_async_remote_copy(
                src_ref=src, dst_ref=dst,
                send_sem=send_sems.at[ch],
                recv_sem=recv_sems.at[ch],
                device_id=(dev,), device_id_type=pl.DeviceIdType.MESH,
            )


        c1A, c1B = [], []
        for j in (0, 1):
            c = rdma(x_ref.at[oct_(k_dg, j), CA], r1A.at[j], 0 + j, p_g)
            c.start()
            c1A.append(c)
            c = rdma(x_ref.at[oct_(k_dg, j), CB], r1B.at[j], 12 + j, p_b)
            c.start()
            c1B.append(c)
        for j in (2, 3):
            c = rdma(x_ref.at[oct_(k_pg, j - 2), CA], r1A.at[j], 0 + j, p_g)
            c.start()
            c1A.append(c)
            c = rdma(x_ref.at[oct_(k_pb, j - 2), CB], r1B.at[j], 12 + j, p_b)
            c.start()
            c1B.append(c)
        out_ref[hk, CA] = bf(hk, CA)
        rows_b = pl.ds(b * q, q)
        rows_2b = pl.ds((2 + b) * q, q)
        out_ref[rows_b, CB] = bf(rows_b, CB)
        out_ref[rows_2b, CB] = bf(rows_2b, CB)

        c2A, c2B = [], []
        for j in (0, 1):
            c1A[j].wait()
            out_ref[oct_(k_pb, j), CA] = (
                out_ref[oct_(k_pb, j), CA] + r1A[j, :, :])
            c = rdma(out_ref.at[oct_(k_pb, j), CA], r2A.at[j], 4 + j, p_b)
            c.start()
            c2A.append(c)
            c1B[j].wait()
            out_ref[oct_(k_pg, j), CB] = (
                out_ref[oct_(k_pg, j), CB] + r1B[j, :, :])
            c = rdma(out_ref.at[oct_(k_pg, j), CB], r2B.at[j], 16 + j, p_g)
            c.start()
            c2B.append(c)

        for j in (2, 3):
            c1A[j].wait()
            out_ref[oct_(k_own, j - 2), CA] = (
                out_ref[oct_(k_own, j - 2), CA] + r1A[j, :, :])
            c1B[j].wait()
            out_ref[oct_(k_own, j - 2), CB] = (
                out_ref[oct_(k_own, j - 2), CB] + r1B[j, :, :])

        c3A, c3B, c4oA, c4oB = [], [], [], []
        for j in (0, 1):
            c2A[j].wait()
            out_ref[oct_(k_own, j), CA] = (
                out_ref[oct_(k_own, j), CA] + r2A[j, :, :])
            ca = rdma(out_ref.at[oct_(k_own, j), CA],
                      out_ref.at[oct_(k_own, j), CA], 6 + j, p_b)
            cb = rdma(out_ref.at[oct_(k_own, j), CA],
                      out_ref.at[oct_(k_own, j), CA], 8 + j, p_g)
            ca.start()
            cb.start()
            c3A.append(ca)
            c4oA.append(cb)
            c2B[j].wait()
            out_ref[oct_(k_own, j), CB] = (
                out_ref[oct_(k_own, j), CB] + r2B[j, :, :])
            ca = rdma(out_ref.at[oct_(k_own, j), CB],
                      out_ref.at[oct_(k_own, j), CB], 18 + j, p_g)
            cb = rdma(out_ref.at[oct_(k_own, j), CB],
                      out_ref.at[oct_(k_own, j), CB], 20 + j, p_b)
            ca.start()
            cb.start()
            c3B.append(ca)
            c4oB.append(cb)

        c4rA, c4rB = [], []
        for j in (0, 1):
            c3A[j].wait()
            c = rdma(out_ref.at[oct_(k_pb, j), CA],
                     out_ref.at[oct_(k_pb, j), CA], 10 + j, p_g)
            c.start()
            c4rA.append(c)
            c3B[j].wait()
            c = rdma(out_ref.at[oct_(k_pg, j), CB],
                     out_ref.at[oct_(k_pg, j), CB], 22 + j, p_b)
            c.start()
            c4rB.append(c)

        for j in (0, 1):
            c4oA[j].wait()
            c4oB[j].wait()
            c4rA[j].wait()
            c4rB[j].wait()

    return pl.pallas_call(
        body,
        out_shape=jax.ShapeDtypeStruct((m, n), jnp.bfloat16),
        in_specs=[pl.BlockSpec(memory_space=pltpu.VMEM)],
        out_specs=pl.BlockSpec(memory_space=pltpu.VMEM),
        scratch_shapes=[
            pltpu.VMEM((4, o, cw), jnp.bfloat16),
            pltpu.VMEM((4, o, cw), jnp.bfloat16),
            pltpu.VMEM((2, o, cw), jnp.bfloat16),
            pltpu.VMEM((2, o, cw), jnp.bfloat16),
            pltpu.SemaphoreType.DMA((24,)),
            pltpu.SemaphoreType.DMA((24,)),
        ],
        compiler_params=pltpu.CompilerParams(collective_id=0),
    )(x.astype(jnp.bfloat16))


# device time: 18139 ns/iter; 1.0111x vs baseline; 1.0111x over previous
import jax
import jax.numpy as jnp
from jax import lax
from jax.experimental import pallas as pl
from jax.experimental.pallas import tpu as pltpu

N_DEV = 4


def kernel(x):
    m, n = x.shape
    q = m // 4
    h = m // 2
    o = q // 2
    cw = n // 2

    CA = pl.ds(0, cw)
    CB = pl.ds(cw, cw)

    def body(x_ref, out_ref, stA, stB, r1A, r1B, r2A, r2B,
             send_sems, recv_sems):
        my = lax.axis_index("i")
        g = my // 2
        b = lax.rem(lax.rem(my, 2) + g, 2)
        p_b = jnp.bitwise_xor(my, 1)
        p_g = jnp.bitwise_xor(my, 3)

        k_own = 2 * g + b
        k_pb = 2 * g + 1 - b
        k_pg = 2 * (1 - g) + b
        k_dg = 2 * (1 - g) + 1 - b
        hk = pl.ds(g * h, h)

        def oct_(k, j):
            return pl.ds(k * q + j * o, o)

        def bf(row_slice, col_slice):
            return x_ref[row_slice, col_slice].astype(jnp.bfloat16)

        barrier_sem = pltpu.get_barrier_semaphore()
        for nbr in (p_b, p_g):
            pl.semaphore_signal(
                barrier_sem, inc=1,
                device_id=(nbr,), device_id_type=pl.DeviceIdType.MESH,
            )
        pl.semaphore_wait(barrier_sem, 2)

        def rdma(src, dst, ch, dev):
            return pltpu.make_async_remote_copy(
                src_ref=src, dst_ref=dst,
                send_sem=send_sems.at[ch],
                recv_sem=recv_sems.at[ch],
                device_id=(dev,), device_id_type=pl.DeviceIdType.MESH,
            )


        c1A, c1B = [], []
        for j in (0, 1):
            stA[j, :, :] = bf(oct_(k_dg, j), CA)
            c = rdma(stA.at[j], r1A.at[j], 0 + j, p_g)
            c.start()
            c1A.append(c)
            stB[j, :, :] = bf(oct_(k_dg, j), CB)
            c = rdma(stB.at[j], r1B.at[j], 12 + j, p_b)
            c.start()
            c1B.append(c)
        for j in (2, 3):
            stA[j, :, :] = bf(oct_(k_pg, j - 2), CA)
            c = rdma(stA.at[j], r1A.at[j], 0 + j, p_g)
            c.start()
            c1A.append(c)
            stB[j, :, :] = bf(oct_(k_pb, j - 2), CB)
            c = rdma(stB.at[j], r1B.at[j], 12 + j, p_b)
            c.start()
            c1B.append(c)
        out_ref[hk, CA] = bf(hk, CA)
        rows_b = pl.ds(b * q, q)
        rows_2b = pl.ds((2 + b) * q, q)
        out_ref[rows_b, CB] = bf(rows_b, CB)
        out_ref[rows_2b, CB] = bf(rows_2b, CB)

        c2A, c2B = [], []
        for j in (0, 1):
            c1A[j].wait()
            out_ref[oct_(k_pb, j), CA] = (
                out_ref[oct_(k_pb, j), CA] + r1A[j, :, :])
            c = rdma(out_ref.at[oct_(k_pb, j), CA], r2A.at[j], 4 + j, p_b)
            c.start()
            c2A.append(c)
            c1B[j].wait()
            out_ref[oct_(k_pg, j), CB] = (
                out_ref[oct_(k_pg, j), CB] + r1B[j, :, :])
            c = rdma(out_ref.at[oct_(k_pg, j), CB], r2B.at[j], 16 + j, p_g)
            c.start()
            c2B.append(c)

        for j in (2, 3):
            c1A[j].wait()
            out_ref[oct_(k_own, j - 2), CA] = (
                out_ref[oct_(k_own, j - 2), CA] + r1A[j, :, :])
            c1B[j].wait()
            out_ref[oct_(k_own, j - 2), CB] = (
                out_ref[oct_(k_own, j - 2), CB] + r1B[j, :, :])

        c3A, c3B, c4oA, c4oB = [], [], [], []
        for j in (0, 1):
            c2A[j].wait()
            out_ref[oct_(k_own, j), CA] = (
                out_ref[oct_(k_own, j), CA] + r2A[j, :, :])
            ca = rdma(out_ref.at[oct_(k_own, j), CA],
                      out_ref.at[oct_(k_own, j), CA], 6 + j, p_b)
            cb = rdma(out_ref.at[oct_(k_own, j), CA],
                      out_ref.at[oct_(k_own, j), CA], 8 + j, p_g)
            ca.start()
            cb.start()
            c3A.append(ca)
            c4oA.append(cb)
            c2B[j].wait()
            out_ref[oct_(k_own, j), CB] = (
                out_ref[oct_(k_own, j), CB] + r2B[j, :, :])
            ca = rdma(out_ref.at[oct_(k_own, j), CB],
                      out_ref.at[oct_(k_own, j), CB], 18 + j, p_g)
            cb = rdma(out_ref.at[oct_(k_own, j), CB],
                      out_ref.at[oct_(k_own, j), CB], 20 + j, p_b)
            ca.start()
            cb.start()
            c3B.append(ca)
            c4oB.append(cb)

        c4rA, c4rB = [], []
        for j in (0, 1):
            c3A[j].wait()
            c = rdma(out_ref.at[oct_(k_pb, j), CA],
                     out_ref.at[oct_(k_pb, j), CA], 10 + j, p_g)
            c.start()
            c4rA.append(c)
            c3B[j].wait()
            c = rdma(out_ref.at[oct_(k_pg, j), CB],
                     out_ref.at[oct_(k_pg, j), CB], 22 + j, p_b)
            c.start()
            c4rB.append(c)

        for j in (0, 1):
            c4oA[j].wait()
            c4oB[j].wait()
            c4rA[j].wait()
            c4rB[j].wait()

    return pl.pallas_call(
        body,
        out_shape=jax.ShapeDtypeStruct((m, n), jnp.bfloat16),
        in_specs=[pl.BlockSpec(memory_space=pltpu.VMEM)],
        out_specs=pl.BlockSpec(memory_space=pltpu.VMEM),
        scratch_shapes=[
            pltpu.VMEM((4, o, cw), jnp.bfloat16),
            pltpu.VMEM((4, o, cw), jnp.bfloat16),
            pltpu.VMEM((4, o, cw), jnp.bfloat16),
            pltpu.VMEM((4, o, cw), jnp.bfloat16),
            pltpu.VMEM((2, o, cw), jnp.bfloat16),
            pltpu.VMEM((2, o, cw), jnp.bfloat16),
            pltpu.SemaphoreType.DMA((24,)),
            pltpu.SemaphoreType.DMA((24,)),
        ],
        compiler_params=pltpu.CompilerParams(collective_id=0),
    )(x)
